# Initial kernel scaffold; baseline (speedup 1.0000x reference)
import jax
import jax.numpy as jnp
from jax.experimental import pallas as pl


def kernel(x, edge_index, edge_features, We1, be1, We2, be2, Wn1, bn1, Wn2, bn2):
    raise NotImplementedError("write your pallas kernel here")


# speedup vs baseline: 2.2191x; 2.2191x over previous
import functools

import jax
import jax.numpy as jnp
from jax import lax
from jax.experimental import pallas as pl
from jax.experimental.pallas import tpu as pltpu
from jax.experimental.pallas import tpu_sc as plsc

N_NODES = 10000
N_EDGES = 320000
NODE_DIM = 128
EDGE_DIM = 16
HIDDEN_DIM = 128

NC = 2
NS = 16
LANES = 16

EDGE_BLOCK_TC = 2000
NODE_BLOCK_TC = 1000
B = 80

E_PER_CORE = N_EDGES // NC
E_PER_TILE = E_PER_CORE // NS
ROWS_PER_TILE = 624
ROWS_REM = N_NODES - NS * ROWS_PER_TILE


def _edge_mlp_body(ef_ref, w1_ref, b1_ref, w2_ref, b2_ref, ew_ref):
    h = jnp.dot(ef_ref[...], w1_ref[...], preferred_element_type=jnp.float32)
    h = h + b1_ref[...]
    h = h * jax.nn.sigmoid(h)
    ew = jnp.dot(h, w2_ref[...], preferred_element_type=jnp.float32)
    ew_ref[...] = ew + b2_ref[...]


def _edge_mlp(ef, We1, be1, We2, be2):
    grid = (N_EDGES // EDGE_BLOCK_TC,)
    return pl.pallas_call(
        _edge_mlp_body,
        grid=grid,
        in_specs=[
            pl.BlockSpec((EDGE_BLOCK_TC, EDGE_DIM), lambda i: (i, 0)),
            pl.BlockSpec((EDGE_DIM, HIDDEN_DIM), lambda i: (0, 0)),
            pl.BlockSpec((1, HIDDEN_DIM), lambda i: (0, 0)),
            pl.BlockSpec((HIDDEN_DIM, NODE_DIM), lambda i: (0, 0)),
            pl.BlockSpec((1, NODE_DIM), lambda i: (0, 0)),
        ],
        out_specs=pl.BlockSpec((EDGE_BLOCK_TC, NODE_DIM), lambda i: (i, 0)),
        out_shape=jax.ShapeDtypeStruct((N_EDGES, NODE_DIM), jnp.float32),
    )(ef, We1, be1.reshape(1, -1), We2, be2.reshape(1, -1))


def _sc_body(x_hbm, src_hbm, dst_hbm, ew_hbm, zero_hbm, out_hbm,
             acc, sidx, didx, xr, ewr, sem):
    c = lax.axis_index("c")
    s = lax.axis_index("s")

    r0 = pl.multiple_of(s * ROWS_PER_TILE, 8)
    pltpu.sync_copy(zero_hbm.at[pl.ds(r0, ROWS_PER_TILE)],
                    acc.at[pl.ds(r0, ROWS_PER_TILE)])

    @pl.when(s == NS - 1)
    def _zero_rem():
        rr = NS * ROWS_PER_TILE
        pltpu.sync_copy(zero_hbm.at[pl.ds(rr, ROWS_REM)],
                        acc.at[pl.ds(rr, ROWS_REM)])

    plsc.subcore_barrier()

    base = c * E_PER_CORE + s * E_PER_TILE

    @pl.loop(0, E_PER_TILE // B)
    def _blk(b):
        e0 = pl.multiple_of(base + b * B, 8)
        pltpu.sync_copy(src_hbm.at[pl.ds(e0, B)], sidx)
        pltpu.sync_copy(dst_hbm.at[pl.ds(e0, B)], didx)
        pltpu.async_copy(x_hbm.at[sidx], xr, sem).wait()
        pltpu.sync_copy(ew_hbm.at[pl.ds(e0, B)], ewr)

        @pl.loop(0, B)
        def _row(r):
            for k in range(0, NODE_DIM, LANES):
                sl = (r, pl.ds(k, LANES))
                xr.at[*sl][...] = xr.at[*sl][...] * ewr.at[*sl][...]

        pltpu.sync_copy(xr, acc.at[didx], add=True)

    plsc.subcore_barrier()
    pltpu.sync_copy(acc.at[pl.ds(r0, ROWS_PER_TILE)],
                    out_hbm.at[c].at[pl.ds(r0, ROWS_PER_TILE)])

    @pl.when(s == NS - 1)
    def _out_rem():
        rr = NS * ROWS_PER_TILE
        pltpu.sync_copy(acc.at[pl.ds(rr, ROWS_REM)],
                        out_hbm.at[c].at[pl.ds(rr, ROWS_REM)])


def _gather_mul_scatter(x, src, dst, ew, zeros):
    mesh = plsc.VectorSubcoreMesh(core_axis_name="c", subcore_axis_name="s")
    fn = pl.kernel(
        _sc_body,
        out_type=jax.ShapeDtypeStruct((NC, N_NODES, NODE_DIM), jnp.float32),
        mesh=mesh,
        scratch_types=[
            pltpu.VMEM_SHARED((N_NODES, NODE_DIM), jnp.float32),
            pltpu.VMEM((B,), jnp.int32),
            pltpu.VMEM((B,), jnp.int32),
            pltpu.VMEM((B, NODE_DIM), jnp.float32),
            pltpu.VMEM((B, NODE_DIM), jnp.float32),
            pltpu.SemaphoreType.DMA,
        ],
    )
    return fn(x, src, dst, ew, zeros)


def _node_mlp_body(p_ref, x_ref, w1_ref, b1_ref, w2_ref, b2_ref, y_ref):
    agg = p_ref[0] + p_ref[1]
    g = jnp.dot(agg, w1_ref[...], preferred_element_type=jnp.float32)
    g = g + b1_ref[...]
    g = g * jax.nn.sigmoid(g)
    o = jnp.dot(g, w2_ref[...], preferred_element_type=jnp.float32)
    y_ref[...] = x_ref[...] + o + b2_ref[...]


def _node_mlp(partials, x, Wn1, bn1, Wn2, bn2):
    grid = (N_NODES // NODE_BLOCK_TC,)
    return pl.pallas_call(
        _node_mlp_body,
        grid=grid,
        in_specs=[
            pl.BlockSpec((NC, NODE_BLOCK_TC, NODE_DIM), lambda i: (0, i, 0)),
            pl.BlockSpec((NODE_BLOCK_TC, NODE_DIM), lambda i: (i, 0)),
            pl.BlockSpec((NODE_DIM, HIDDEN_DIM), lambda i: (0, 0)),
            pl.BlockSpec((1, HIDDEN_DIM), lambda i: (0, 0)),
            pl.BlockSpec((HIDDEN_DIM, NODE_DIM), lambda i: (0, 0)),
            pl.BlockSpec((1, NODE_DIM), lambda i: (0, 0)),
        ],
        out_specs=pl.BlockSpec((NODE_BLOCK_TC, NODE_DIM), lambda i: (i, 0)),
        out_shape=jax.ShapeDtypeStruct((N_NODES, NODE_DIM), jnp.float32),
    )(partials, x, Wn1, bn1.reshape(1, -1), Wn2, bn2.reshape(1, -1))


def kernel(x, edge_index, edge_features, We1, be1, We2, be2, Wn1, bn1, Wn2, bn2):
    src = edge_index[0].astype(jnp.int32)
    dst = edge_index[1].astype(jnp.int32)
    ew = _edge_mlp(edge_features, We1, be1, We2, be2)
    zeros = jnp.zeros((N_NODES, NODE_DIM), jnp.float32)
    partials = _gather_mul_scatter(x, src, dst, ew, zeros)
    return _node_mlp(partials, x, Wn1, bn1, Wn2, bn2)


# speedup vs baseline: 3.6024x; 1.6233x over previous
import functools

import jax
import jax.numpy as jnp
from jax import lax
from jax.experimental import pallas as pl
from jax.experimental.pallas import tpu as pltpu
from jax.experimental.pallas import tpu_sc as plsc

N_NODES = 10000
N_EDGES = 320000
NODE_DIM = 128
EDGE_DIM = 16
HIDDEN_DIM = 128

NC = 2
NS = 16
LANES = 16

EDGE_BLOCK_TC = 2000
NODE_BLOCK_TC = 1000
B = 40

E_PER_CORE = N_EDGES // NC
E_PER_TILE = E_PER_CORE // NS
ROWS_PER_TILE = 624
ROWS_REM = N_NODES - NS * ROWS_PER_TILE


def _edge_mlp_body(ef_ref, w1_ref, b1_ref, w2_ref, b2_ref, ew_ref):
    h = jnp.dot(ef_ref[...], w1_ref[...], preferred_element_type=jnp.float32)
    h = h + b1_ref[...]
    h = h * jax.nn.sigmoid(h)
    ew = jnp.dot(h, w2_ref[...], preferred_element_type=jnp.float32)
    ew_ref[...] = ew + b2_ref[...]


def _edge_mlp(ef, We1, be1, We2, be2):
    grid = (N_EDGES // EDGE_BLOCK_TC,)
    return pl.pallas_call(
        _edge_mlp_body,
        grid=grid,
        in_specs=[
            pl.BlockSpec((EDGE_BLOCK_TC, EDGE_DIM), lambda i: (i, 0)),
            pl.BlockSpec((EDGE_DIM, HIDDEN_DIM), lambda i: (0, 0)),
            pl.BlockSpec((1, HIDDEN_DIM), lambda i: (0, 0)),
            pl.BlockSpec((HIDDEN_DIM, NODE_DIM), lambda i: (0, 0)),
            pl.BlockSpec((1, NODE_DIM), lambda i: (0, 0)),
        ],
        out_specs=pl.BlockSpec((EDGE_BLOCK_TC, NODE_DIM), lambda i: (i, 0)),
        out_shape=jax.ShapeDtypeStruct((N_EDGES, NODE_DIM), jnp.float32),
    )(ef, We1, be1.reshape(1, -1), We2, be2.reshape(1, -1))


NB = E_PER_TILE // B
NBUF = 4
NIB = 8


def _sc_body(x_hbm, idx_hbm, ew_hbm, zero_hbm, out_hbm,
             acc, i0, i1, i2, i3, i4, i5, i6, i7,
             xr0, xr1, xr2, xr3, ew0, ew1, ew2, ew3,
             is0, is1, is2, is3, is4, is5, is6, is7,
             g0, g1, g2, g3,
             e0s, e1s, e2s, e3s, s0, s1, s2, s3):
    c = lax.axis_index("c")
    s = lax.axis_index("s")
    ib = (i0, i1, i2, i3, i4, i5, i6, i7)
    xr = (xr0, xr1, xr2, xr3)
    ewr = (ew0, ew1, ew2, ew3)
    isem = (is0, is1, is2, is3, is4, is5, is6, is7)
    gsem = (g0, g1, g2, g3)
    esem = (e0s, e1s, e2s, e3s)
    ssem = (s0, s1, s2, s3)

    r0 = pl.multiple_of(s * ROWS_PER_TILE, 8)
    pltpu.sync_copy(zero_hbm.at[pl.ds(r0, ROWS_PER_TILE)],
                    acc.at[pl.ds(r0, ROWS_PER_TILE)])

    @pl.when(s == NS - 1)
    def _zero_rem():
        rr = NS * ROWS_PER_TILE
        pltpu.sync_copy(zero_hbm.at[pl.ds(rr, ROWS_REM)],
                        acc.at[pl.ds(rr, ROWS_REM)])

    plsc.subcore_barrier()

    w = c * NS + s
    ebase = w * E_PER_TILE

    def load_idx(b, si):
        pltpu.async_copy(idx_hbm.at[w].at[b], ib[si], isem[si])

    def wait_idx(b, si):
        pltpu.make_async_copy(idx_hbm.at[w].at[b], ib[si], isem[si]).wait()

    def load_data(b, si, sd):
        eoff = pl.multiple_of(ebase + b * B, 8)
        pltpu.async_copy(x_hbm.at[ib[si].at[0]], xr[sd], gsem[sd])
        pltpu.async_copy(ew_hbm.at[pl.ds(eoff, B)], ewr[sd], esem[sd])

    def wait_data(b, si, sd):
        pltpu.make_async_copy(x_hbm.at[ib[si].at[0]], xr[sd],
                              gsem[sd]).wait()
        eoff = pl.multiple_of(ebase + b * B, 8)
        pltpu.make_async_copy(ew_hbm.at[pl.ds(eoff, B)], ewr[sd],
                              esem[sd]).wait()

    def mul(sd):
        @pl.loop(0, B, step=2)
        def _row(r):
            for dr in range(2):
                for k in range(0, NODE_DIM, LANES):
                    sl = (r + dr, pl.ds(k, LANES))
                    xr[sd].at[*sl][...] = (xr[sd].at[*sl][...]
                                           * ewr[sd].at[*sl][...])

    def scatter(si, sd):
        pltpu.async_copy(xr[sd], acc.at[ib[si].at[1]], ssem[sd], add=True)

    def wait_scatter(si, sd):
        pltpu.make_async_copy(xr[sd], acc.at[ib[si].at[1]], ssem[sd]).wait()

    for b in (0, 1, 2):
        load_idx(b, b % NIB)
    for b in (0, 1):
        wait_idx(b, b % NIB)
        load_data(b, b % NIB, b % NBUF)

    def stage(b, st):
        inb = b + 3
        nb = b + 2

        @pl.when(inb < NB)
        def _pf_idx():
            load_idx(inb, (st + 3) % NIB)

        @pl.when(nb < NB)
        def _pf_data():
            @pl.when(nb >= NBUF)
            def _drain():
                wait_scatter((st - 2) % NIB, (st - 2) % NBUF)

            wait_idx(nb, (st + 2) % NIB)
            load_data(nb, (st + 2) % NIB, (st + 2) % NBUF)

        @pl.when(b < NB)
        def _work():
            wait_data(b, st % NIB, st % NBUF)
            mul(st % NBUF)
            scatter(st % NIB, st % NBUF)

    n_groups = (NB + NIB - 1) // NIB

    @pl.loop(0, n_groups)
    def _grp(k):
        kb = k * NIB
        for st in range(NIB):
            stage(kb + st, st)

    for b in range(NB - NBUF, NB):
        wait_scatter(b % NIB, b % NBUF)

    plsc.subcore_barrier()
    pltpu.sync_copy(acc.at[pl.ds(r0, ROWS_PER_TILE)],
                    out_hbm.at[c].at[pl.ds(r0, ROWS_PER_TILE)])

    @pl.when(s == NS - 1)
    def _out_rem():
        rr = NS * ROWS_PER_TILE
        pltpu.sync_copy(acc.at[pl.ds(rr, ROWS_REM)],
                        out_hbm.at[c].at[pl.ds(rr, ROWS_REM)])


def _gather_mul_scatter(x, src, dst, ew, zeros):
    mesh = plsc.VectorSubcoreMesh(core_axis_name="c", subcore_axis_name="s")
    dma = pltpu.SemaphoreType.DMA
    fn = pl.kernel(
        _sc_body,
        out_type=jax.ShapeDtypeStruct((NC, N_NODES, NODE_DIM), jnp.float32),
        mesh=mesh,
        scratch_types=(
            [pltpu.VMEM_SHARED((N_NODES, NODE_DIM), jnp.float32)]
            + [pltpu.VMEM((2, B), jnp.int32) for _ in range(NIB)]
            + [pltpu.VMEM((B, NODE_DIM), jnp.float32) for _ in range(2 * NBUF)]
            + [dma for _ in range(NIB + 3 * NBUF)]
        ),
    )
    idx = jnp.stack([src.reshape(NC * NS, NB, B),
                     dst.reshape(NC * NS, NB, B)], axis=2)
    return fn(x, idx, ew, zeros)


def _node_mlp_body(p_ref, x_ref, w1_ref, b1_ref, w2_ref, b2_ref, y_ref):
    agg = p_ref[0] + p_ref[1]
    g = jnp.dot(agg, w1_ref[...], preferred_element_type=jnp.float32)
    g = g + b1_ref[...]
    g = g * jax.nn.sigmoid(g)
    o = jnp.dot(g, w2_ref[...], preferred_element_type=jnp.float32)
    y_ref[...] = x_ref[...] + o + b2_ref[...]


def _node_mlp(partials, x, Wn1, bn1, Wn2, bn2):
    grid = (N_NODES // NODE_BLOCK_TC,)
    return pl.pallas_call(
        _node_mlp_body,
        grid=grid,
        in_specs=[
            pl.BlockSpec((NC, NODE_BLOCK_TC, NODE_DIM), lambda i: (0, i, 0)),
            pl.BlockSpec((NODE_BLOCK_TC, NODE_DIM), lambda i: (i, 0)),
            pl.BlockSpec((NODE_DIM, HIDDEN_DIM), lambda i: (0, 0)),
            pl.BlockSpec((1, HIDDEN_DIM), lambda i: (0, 0)),
            pl.BlockSpec((HIDDEN_DIM, NODE_DIM), lambda i: (0, 0)),
            pl.BlockSpec((1, NODE_DIM), lambda i: (0, 0)),
        ],
        out_specs=pl.BlockSpec((NODE_BLOCK_TC, NODE_DIM), lambda i: (i, 0)),
        out_shape=jax.ShapeDtypeStruct((N_NODES, NODE_DIM), jnp.float32),
    )(partials, x, Wn1, bn1.reshape(1, -1), Wn2, bn2.reshape(1, -1))


def kernel(x, edge_index, edge_features, We1, be1, We2, be2, Wn1, bn1, Wn2, bn2):
    src = edge_index[0].astype(jnp.int32)
    dst = edge_index[1].astype(jnp.int32)
    ew = _edge_mlp(edge_features, We1, be1, We2, be2)
    zeros = jnp.zeros((N_NODES, NODE_DIM), jnp.float32)
    partials = _gather_mul_scatter(x, src, dst, ew, zeros)
    return _node_mlp(partials, x, Wn1, bn1, Wn2, bn2)


# speedup vs baseline: 4.2546x; 1.1810x over previous
import functools

import jax
import jax.numpy as jnp
from jax import lax
from jax.experimental import pallas as pl
from jax.experimental.pallas import tpu as pltpu
from jax.experimental.pallas import tpu_sc as plsc

N_NODES = 10000
N_EDGES = 320000
NODE_DIM = 128
EDGE_DIM = 16
HIDDEN_DIM = 128

NC = 2
NS = 16
LANES = 16

EDGE_BLOCK_TC = 4000
NODE_BLOCK_TC = 1000
B = 40

E_PER_CORE = N_EDGES // NC
E_PER_TILE = E_PER_CORE // NS
ROWS_PER_TILE = 624
ROWS_REM = N_NODES - NS * ROWS_PER_TILE


def _edge_mlp_body(ef_ref, w1_ref, b1_ref, w2_ref, b2_ref, ew_ref):
    h = jnp.dot(ef_ref[...], w1_ref[...], preferred_element_type=jnp.float32)
    h = h + b1_ref[...]
    h = h * jax.nn.sigmoid(h)
    ew = jnp.dot(h, w2_ref[...], preferred_element_type=jnp.float32)
    ew_ref[...] = ew + b2_ref[...]


def _edge_mlp(ef, We1, be1, We2, be2):
    grid = (N_EDGES // EDGE_BLOCK_TC,)
    return pl.pallas_call(
        _edge_mlp_body,
        grid=grid,
        in_specs=[
            pl.BlockSpec((EDGE_BLOCK_TC, EDGE_DIM), lambda i: (i, 0)),
            pl.BlockSpec((EDGE_DIM, HIDDEN_DIM), lambda i: (0, 0)),
            pl.BlockSpec((1, HIDDEN_DIM), lambda i: (0, 0)),
            pl.BlockSpec((HIDDEN_DIM, NODE_DIM), lambda i: (0, 0)),
            pl.BlockSpec((1, NODE_DIM), lambda i: (0, 0)),
        ],
        out_specs=pl.BlockSpec((EDGE_BLOCK_TC, NODE_DIM), lambda i: (i, 0)),
        out_shape=jax.ShapeDtypeStruct((N_EDGES, NODE_DIM), jnp.float32),
    )(ef, We1, be1.reshape(1, -1), We2, be2.reshape(1, -1))


NB = E_PER_TILE // B
NBUF = 4
NIB = 8


def _sc_body(x_hbm, src_hbm, dst_hbm, ew_hbm, zero_hbm, out_hbm,
             acc, i0, i1, i2, i3, i4, i5, i6, i7,
             xr0, xr1, xr2, xr3, ew0, ew1, ew2, ew3,
             is0, is1, is2, is3, is4, is5, is6, is7,
             g0, g1, g2, g3,
             e0s, e1s, e2s, e3s, s0, s1, s2, s3):
    c = lax.axis_index("c")
    s = lax.axis_index("s")
    ib = (i0, i1, i2, i3, i4, i5, i6, i7)
    xr = (xr0, xr1, xr2, xr3)
    ewr = (ew0, ew1, ew2, ew3)
    isem = (is0, is1, is2, is3, is4, is5, is6, is7)
    gsem = (g0, g1, g2, g3)
    esem = (e0s, e1s, e2s, e3s)
    ssem = (s0, s1, s2, s3)

    r0 = pl.multiple_of(s * ROWS_PER_TILE, 8)
    pltpu.sync_copy(zero_hbm.at[pl.ds(r0, ROWS_PER_TILE)],
                    acc.at[pl.ds(r0, ROWS_PER_TILE)])

    @pl.when(s == NS - 1)
    def _zero_rem():
        rr = NS * ROWS_PER_TILE
        pltpu.sync_copy(zero_hbm.at[pl.ds(rr, ROWS_REM)],
                        acc.at[pl.ds(rr, ROWS_REM)])

    plsc.subcore_barrier()

    w = c * NS + s
    ebase = w * E_PER_TILE

    def load_idx(b, si):
        pltpu.async_copy(src_hbm.at[w].at[b], ib[si].at[0], isem[si])
        pltpu.async_copy(dst_hbm.at[w].at[b], ib[si].at[1], isem[si])

    def wait_idx(b, si):
        pltpu.make_async_copy(src_hbm.at[w].at[b], ib[si].at[0],
                              isem[si]).wait()
        pltpu.make_async_copy(dst_hbm.at[w].at[b], ib[si].at[1],
                              isem[si]).wait()

    def load_data(b, si, sd):
        eoff = pl.multiple_of(ebase + b * B, 8)
        pltpu.async_copy(x_hbm.at[ib[si].at[0]], xr[sd], gsem[sd])
        pltpu.async_copy(ew_hbm.at[pl.ds(eoff, B)], ewr[sd], esem[sd])

    def wait_data(b, si, sd):
        pltpu.make_async_copy(x_hbm.at[ib[si].at[0]], xr[sd],
                              gsem[sd]).wait()
        eoff = pl.multiple_of(ebase + b * B, 8)
        pltpu.make_async_copy(ew_hbm.at[pl.ds(eoff, B)], ewr[sd],
                              esem[sd]).wait()

    def mul(sd):
        @pl.loop(0, B, step=2)
        def _row(r):
            for dr in range(2):
                for k in range(0, NODE_DIM, LANES):
                    sl = (r + dr, pl.ds(k, LANES))
                    xr[sd].at[*sl][...] = (xr[sd].at[*sl][...]
                                           * ewr[sd].at[*sl][...])

    def scatter(si, sd):
        pltpu.async_copy(xr[sd], acc.at[ib[si].at[1]], ssem[sd], add=True)

    def wait_scatter(si, sd):
        pltpu.make_async_copy(xr[sd], acc.at[ib[si].at[1]], ssem[sd]).wait()

    for b in (0, 1, 2):
        load_idx(b, b % NIB)
    for b in (0, 1):
        wait_idx(b, b % NIB)
        load_data(b, b % NIB, b % NBUF)

    def stage(b, st):
        inb = b + 3
        nb = b + 2

        @pl.when(inb < NB)
        def _pf_idx():
            load_idx(inb, (st + 3) % NIB)

        @pl.when(nb < NB)
        def _pf_data():
            @pl.when(nb >= NBUF)
            def _drain():
                wait_scatter((st - 2) % NIB, (st - 2) % NBUF)

            wait_idx(nb, (st + 2) % NIB)
            load_data(nb, (st + 2) % NIB, (st + 2) % NBUF)

        @pl.when(b < NB)
        def _work():
            wait_data(b, st % NIB, st % NBUF)
            mul(st % NBUF)
            scatter(st % NIB, st % NBUF)

    n_groups = (NB + NIB - 1) // NIB

    @pl.loop(0, n_groups)
    def _grp(k):
        kb = k * NIB
        for st in range(NIB):
            stage(kb + st, st)

    for b in range(NB - NBUF, NB):
        wait_scatter(b % NIB, b % NBUF)

    plsc.subcore_barrier()
    pltpu.sync_copy(acc.at[pl.ds(r0, ROWS_PER_TILE)],
                    out_hbm.at[c].at[pl.ds(r0, ROWS_PER_TILE)])

    @pl.when(s == NS - 1)
    def _out_rem():
        rr = NS * ROWS_PER_TILE
        pltpu.sync_copy(acc.at[pl.ds(rr, ROWS_REM)],
                        out_hbm.at[c].at[pl.ds(rr, ROWS_REM)])


def _gather_mul_scatter(x, src, dst, ew, zeros):
    mesh = plsc.VectorSubcoreMesh(core_axis_name="c", subcore_axis_name="s")
    dma = pltpu.SemaphoreType.DMA
    fn = pl.kernel(
        _sc_body,
        out_type=jax.ShapeDtypeStruct((NC, N_NODES, NODE_DIM), jnp.float32),
        mesh=mesh,
        scratch_types=(
            [pltpu.VMEM_SHARED((N_NODES, NODE_DIM), jnp.float32)]
            + [pltpu.VMEM((2, B), jnp.int32) for _ in range(NIB)]
            + [pltpu.VMEM((B, NODE_DIM), jnp.float32) for _ in range(2 * NBUF)]
            + [dma for _ in range(NIB + 3 * NBUF)]
        ),
    )
    return fn(x, src.reshape(NC * NS, NB, B), dst.reshape(NC * NS, NB, B),
              ew, zeros)


def _node_mlp_body(p_ref, x_ref, w1_ref, b1_ref, w2_ref, b2_ref, y_ref):
    agg = p_ref[0] + p_ref[1]
    g = jnp.dot(agg, w1_ref[...], preferred_element_type=jnp.float32)
    g = g + b1_ref[...]
    g = g * jax.nn.sigmoid(g)
    o = jnp.dot(g, w2_ref[...], preferred_element_type=jnp.float32)
    y_ref[...] = x_ref[...] + o + b2_ref[...]


def _node_mlp(partials, x, Wn1, bn1, Wn2, bn2):
    grid = (N_NODES // NODE_BLOCK_TC,)
    return pl.pallas_call(
        _node_mlp_body,
        grid=grid,
        in_specs=[
            pl.BlockSpec((NC, NODE_BLOCK_TC, NODE_DIM), lambda i: (0, i, 0)),
            pl.BlockSpec((NODE_BLOCK_TC, NODE_DIM), lambda i: (i, 0)),
            pl.BlockSpec((NODE_DIM, HIDDEN_DIM), lambda i: (0, 0)),
            pl.BlockSpec((1, HIDDEN_DIM), lambda i: (0, 0)),
            pl.BlockSpec((HIDDEN_DIM, NODE_DIM), lambda i: (0, 0)),
            pl.BlockSpec((1, NODE_DIM), lambda i: (0, 0)),
        ],
        out_specs=pl.BlockSpec((NODE_BLOCK_TC, NODE_DIM), lambda i: (i, 0)),
        out_shape=jax.ShapeDtypeStruct((N_NODES, NODE_DIM), jnp.float32),
    )(partials, x, Wn1, bn1.reshape(1, -1), Wn2, bn2.reshape(1, -1))


def kernel(x, edge_index, edge_features, We1, be1, We2, be2, Wn1, bn1, Wn2, bn2):
    src = edge_index[0].astype(jnp.int32)
    dst = edge_index[1].astype(jnp.int32)
    ew = _edge_mlp(edge_features, We1, be1, We2, be2)
    zeros = jnp.zeros((N_NODES, NODE_DIM), jnp.float32)
    partials = _gather_mul_scatter(x, src, dst, ew, zeros)
    return _node_mlp(partials, x, Wn1, bn1, Wn2, bn2)
